# Initial kernel scaffold; baseline (speedup 1.0000x reference)
import jax
import jax.numpy as jnp
from jax.experimental import pallas as pl


def kernel(params, x, edge_index, batch):
    raise NotImplementedError("write your pallas kernel here")


# speedup vs baseline: 2.2589x; 2.2589x over previous
import functools

import jax
import jax.numpy as jnp
from jax import lax
from jax.experimental import pallas as pl
from jax.experimental.pallas import tpu as pltpu
from jax.experimental.pallas import tpu_sc as plsc

NC, NS = 2, 16
NW = NC * NS
L = 16

N_PAD = 10240
E_PAD = 163840
HALF = N_PAD // 2
ACC_ROWS = HALF + 8
DEG_ROWS = 10496
T = 8
D = 256
G = 128
NBLK = N_PAD // 256
K = 128

_mesh = functools.partial(
    plsc.VectorSubcoreMesh, core_axis_name="c", subcore_axis_name="s")


def _sc_pre(emb, xflat, loc2):
    TOK = xflat.shape[0]
    tpw = TOK // NW
    ept = E_PAD // NS
    DACC = HALF + 128
    zrows = DACC // NS
    orows = HALF // NS
    DW = 128

    @functools.partial(
        pl.kernel,
        mesh=_mesh(),
        out_type=(jax.ShapeDtypeStruct((TOK, D), jnp.float32),
                  jax.ShapeDtypeStruct((NC * HALF, DW), jnp.float32)),
        scratch_types=[pltpu.VMEM((K,), jnp.int32),
                       pltpu.VMEM((K, D), jnp.float32),
                       pltpu.VMEM((K,), jnp.int32),
                       pltpu.VMEM((K, DW), jnp.float32),
                       pltpu.VMEM((41, DW), jnp.float32),
                       pltpu.VMEM_SHARED((DACC, DW), jnp.float32),
                       pltpu.SemaphoreType.DMA],
    )
    def k(emb_h, xf_h, loc_h, tok_h, degp_h, idxv, rows, locv, vbuf, zbuf,
          acc, sem):
        c = lax.axis_index("c")
        s = lax.axis_index("s")
        w = c * NS + s

        one16 = jnp.where(lax.iota(jnp.int32, L) == 0, 1.0, 0.0)
        zero16 = jnp.zeros((L,), jnp.float32)

        @pl.loop(0, K)
        def _fill(i):
            for l in range(DW // L):
                vbuf[i, pl.ds(l * L, L)] = one16 if l == 0 else zero16

        @pl.loop(0, 41)
        def _zfill(i):
            for l in range(DW // L):
                zbuf[i, pl.ds(l * L, L)] = zero16

        for z in range(zrows // 41):
            pltpu.sync_copy(zbuf, acc.at[pl.ds(s * zrows + z * 41, 41)])
        plsc.subcore_barrier()

        eb = s * ept

        @pl.loop(0, ept // K)
        def _deg(j):
            pltpu.sync_copy(loc_h.at[c, pl.ds(eb + j * K, K)], locv)
            pltpu.sync_copy(vbuf, acc.at[locv], add=True)

        tb = w * tpw

        @pl.loop(0, tpw // K)
        def _emb(j):
            base = tb + j * K
            pltpu.sync_copy(xf_h.at[pl.ds(base, K)], idxv)
            pltpu.async_copy(emb_h.at[idxv], rows, sem).wait()
            pltpu.sync_copy(rows, tok_h.at[pl.ds(base, K), :])

        plsc.subcore_barrier()
        pltpu.sync_copy(acc.at[pl.ds(s * orows, orows)],
                        degp_h.at[pl.ds(c * HALF + s * orows, orows), :])

    return k(emb, xflat, loc2)


def _sc_msgpass(hN2, src2, loc2):
    DH = D // 2
    ept = E_PAD // NS
    rpt = HALF // NS

    @functools.partial(
        pl.kernel,
        mesh=_mesh(),
        out_type=jax.ShapeDtypeStruct((2 * N_PAD, DH), jnp.float32),
        scratch_types=[pltpu.VMEM((K,), jnp.int32),
                       pltpu.VMEM((K,), jnp.int32),
                       pltpu.VMEM((K, DH), jnp.float32),
                       pltpu.VMEM_SHARED((ACC_ROWS, DH), jnp.float32),
                       pltpu.SemaphoreType.DMA],
    )
    def k(hN_h, src_h, loc_h, out_h, srcv, locv, rows, acc, sem):
        c = lax.axis_index("c")
        s = lax.axis_index("s")
        gbase = c * HALF + s * rpt
        eb = s * ept

        for f in range(2):
            fb = f * N_PAD
            pltpu.sync_copy(hN_h.at[pl.ds(fb + gbase, rpt), :],
                            acc.at[pl.ds(s * rpt, rpt)])
            plsc.subcore_barrier()

            @pl.loop(0, ept // K)
            def _edges(j):
                b = eb + j * K
                pltpu.sync_copy(src_h.at[f, pl.ds(b, K)], srcv)
                pltpu.sync_copy(loc_h.at[c, pl.ds(b, K)], locv)
                pltpu.async_copy(hN_h.at[srcv], rows, sem).wait()
                pltpu.sync_copy(rows, acc.at[locv], add=True)

            plsc.subcore_barrier()
            pltpu.sync_copy(acc.at[pl.ds(s * rpt, rpt)],
                            out_h.at[pl.ds(fb + gbase, rpt), :])

    return k(hN2, src2, loc2)


def _dot(a, b):
    return jax.lax.dot_general(a, b, (((1,), (0,)), ((), ())),
                               preferred_element_type=jnp.float32)


def _prep_kernel(dst_ref, src_ref, loc_ref, src2_ref):
    d = dst_ref[...]
    sv = src_ref[...]
    for c in range(NC):
        lo = c * HALF
        inr = jnp.logical_and(d >= lo, d < lo + HALF)
        loc_ref[c] = jnp.where(inr, d - lo, HALF)
    for f in range(2):
        src2_ref[f] = sv + f * N_PAD


def _tc_prep(dst2d, src2d):
    return pl.pallas_call(
        _prep_kernel,
        out_shape=[jax.ShapeDtypeStruct((NC,) + dst2d.shape, jnp.int32),
                   jax.ShapeDtypeStruct((2,) + dst2d.shape, jnp.int32)],
    )(dst2d, src2d)


def _enc_kernel(tok_ref, x_ref, degp_ref, stw_ref, stb_ref, inw_ref,
                resw_ref, resb_ref, hNin_ref, res0_ref, norm8_ref):
    xb = x_ref[...]
    mask = (xb != 0).astype(jnp.float32)
    tok = tok_ref[...]
    msum = jnp.sum(tok * mask[:, :, None], axis=1)
    cnt = jnp.sum(mask, axis=1, keepdims=True)
    node = msum / jnp.maximum(cnt, 1.0)
    node = _dot(node, stw_ref[...]) + stb_ref[...]

    deg = degp_ref[:, :1] + 1.0
    norm = jax.lax.rsqrt(jnp.maximum(deg, 1.0))
    norm8_ref[...] = jnp.broadcast_to(norm, norm8_ref.shape)

    hNin = _dot(node, inw_ref[...]) * norm
    hNin_ref[0] = hNin[:, :D // 2]
    hNin_ref[1] = hNin[:, D // 2:]
    res0_ref[...] = _dot(node, resw_ref[...]) + resb_ref[...]


def _tc_encoder(tok3, xp, degp, st_W, st_b, in_W, res_W, res_b):
    full = lambda shape: pl.BlockSpec(shape, lambda i: (0,) * len(shape))
    return pl.pallas_call(
        _enc_kernel,
        grid=(NBLK,),
        in_specs=[
            pl.BlockSpec((256, T, D), lambda i: (i, 0, 0)),
            pl.BlockSpec((256, T), lambda i: (i, 0)),
            pl.BlockSpec((256, 128), lambda i: (i, 0)),
            full((D, D)), full((1, D)), full((D, D)), full((D, D)),
            full((1, D)),
        ],
        out_specs=[
            pl.BlockSpec((2, 256, D // 2), lambda i: (0, i, 0)),
            pl.BlockSpec((256, D), lambda i: (i, 0)),
            pl.BlockSpec((256, T), lambda i: (i, 0)),
        ],
        out_shape=[
            jax.ShapeDtypeStruct((2, N_PAD, D // 2), jnp.float32),
            jax.ShapeDtypeStruct((N_PAD, D), jnp.float32),
            jax.ShapeDtypeStruct((N_PAD, T), jnp.float32),
        ],
    )(tok3, xp, degp, st_W, st_b.reshape(1, D), in_W, res_W,
      res_b.reshape(1, D))


def _cat(s2):
    return jnp.concatenate([s2[0], s2[1]], axis=1)


def _split_store(ref, v):
    ref[0] = v[:, :D // 2]
    ref[1] = v[:, D // 2:]


def _post0_kernel(sums_ref, norm8_ref, res0_ref, inb_ref, w_ref,
                  h_ref, hN_ref):
    nrm = norm8_ref[:, :1]
    h = jax.nn.relu(nrm * _cat(sums_ref[...]) + inb_ref[...]) + res0_ref[...]
    h_ref[...] = h
    _split_store(hN_ref, _dot(h, w_ref[...]) * nrm)


_sums_blk = lambda: pl.BlockSpec((2, 256, D // 2), lambda i: (0, i, 0))
_hN_shape = lambda: jax.ShapeDtypeStruct((2, N_PAD, D // 2), jnp.float32)


def _tc_post0(sums2, norm8, res0, in_b, W0):
    full = lambda shape: pl.BlockSpec(shape, lambda i: (0,) * len(shape))
    blk = pl.BlockSpec((256, D), lambda i: (i, 0))
    return pl.pallas_call(
        _post0_kernel,
        grid=(NBLK,),
        in_specs=[_sums_blk(), pl.BlockSpec((256, T), lambda i: (i, 0)), blk,
                  full((1, D)), full((D, D))],
        out_specs=[blk, _sums_blk()],
        out_shape=[jax.ShapeDtypeStruct((N_PAD, D), jnp.float32),
                   _hN_shape()],
    )(sums2, norm8, res0, in_b.reshape(1, D), W0)


def _layer_body(sums_ref, norm8_ref, hprev_ref, b_ref, g_ref, beta_ref):
    nrm = norm8_ref[:, :1]
    u = nrm * _cat(sums_ref[...]) + b_ref[...]
    mu = jnp.mean(u, axis=1, keepdims=True)
    var = jnp.mean(jnp.square(u - mu), axis=1, keepdims=True)
    u = (u - mu) * jax.lax.rsqrt(var + 1e-5) * g_ref[...] + beta_ref[...]
    return jax.nn.relu(u) + hprev_ref[...], nrm


def _layer_kernel(sums_ref, norm8_ref, hprev_ref, b_ref, g_ref, beta_ref,
                  w_ref, h_ref, hN_ref):
    h, nrm = _layer_body(sums_ref, norm8_ref, hprev_ref, b_ref, g_ref,
                         beta_ref)
    h_ref[...] = h
    _split_store(hN_ref, _dot(h, w_ref[...]) * nrm)


def _tc_layer(sums2, norm8, hprev, b, g, beta, Wnext):
    full = lambda shape: pl.BlockSpec(shape, lambda i: (0,) * len(shape))
    blk = pl.BlockSpec((256, D), lambda i: (i, 0))
    return pl.pallas_call(
        _layer_kernel,
        grid=(NBLK,),
        in_specs=[_sums_blk(), pl.BlockSpec((256, T), lambda i: (i, 0)), blk,
                  full((1, D)), full((1, D)), full((1, D)), full((D, D))],
        out_specs=[blk, _sums_blk()],
        out_shape=[jax.ShapeDtypeStruct((N_PAD, D), jnp.float32),
                   _hN_shape()],
    )(sums2, norm8, hprev, b.reshape(1, D), g.reshape(1, D),
      beta.reshape(1, D), Wnext)


def _last_kernel(sums_ref, norm8_ref, hprev_ref, b_ref, g_ref, beta_ref,
                 gw1_ref, gb1_ref, gw2_ref, gb2_ref, h_ref, gate8_ref):
    h, _ = _layer_body(sums_ref, norm8_ref, hprev_ref, b_ref, g_ref,
                       beta_ref)
    h_ref[...] = h
    g1 = jax.nn.relu(_dot(h, gw1_ref[...]) + gb1_ref[...])
    gate8_ref[...] = _dot(g1, gw2_ref[...]) + gb2_ref[...]


def _tc_last(sums2, norm8, hprev, b, g, beta, gw1, gb1, gw2, gb2):
    full = lambda shape: pl.BlockSpec(shape, lambda i: (0,) * len(shape))
    blk = pl.BlockSpec((256, D), lambda i: (i, 0))
    return pl.pallas_call(
        _last_kernel,
        grid=(NBLK,),
        in_specs=[_sums_blk(), pl.BlockSpec((256, T), lambda i: (i, 0)), blk,
                  full((1, D)), full((1, D)), full((1, D)),
                  full((D, D // 2)), full((1, D // 2)),
                  full((D // 2, T)), full((1, T))],
        out_specs=[blk, pl.BlockSpec((256, T), lambda i: (i, 0))],
        out_shape=[jax.ShapeDtypeStruct((N_PAD, D), jnp.float32),
                   jax.ShapeDtypeStruct((N_PAD, T), jnp.float32)],
    )(sums2, norm8, hprev, b.reshape(1, D), g.reshape(1, D),
      beta.reshape(1, D), gw1, gb1.reshape(1, D // 2),
      jnp.pad(gw2, ((0, 0), (0, T - 1))),
      jnp.broadcast_to(gb2.reshape(1, 1), (1, T)))


def _pool_kernel(h_ref, gate8_ref, batch8_ref, out_ref, acc, m, sden):
    i = pl.program_id(0)

    @pl.when(i == 0)
    def _init():
        acc[...] = jnp.zeros_like(acc)
        m[...] = jnp.full_like(m, -jnp.inf)
        sden[...] = jnp.zeros_like(sden)

    batch = batch8_ref[:, :1]
    gate = gate8_ref[:, :1]
    giota = lax.broadcasted_iota(jnp.int32, (256, G), 1)
    P = batch == giota
    Pf = P.astype(jnp.float32)

    m_old = m[...]
    cand = jnp.max(jnp.where(P, gate, -jnp.inf), axis=0, keepdims=True)
    m_new = jnp.maximum(m_old, cand)
    m[...] = m_new

    finite = m_new > -jnp.inf
    scale = jnp.exp(jnp.where(finite, m_old - m_new, 0.0))

    m_node = jnp.sum(jnp.where(P, jnp.where(finite, m_new, 0.0), 0.0),
                     axis=1, keepdims=True)
    valid = batch < G
    e = jnp.where(valid, jnp.exp(gate - m_node), 0.0)

    sden[...] = sden[...] * scale + jnp.sum(Pf * e, axis=0, keepdims=True)
    he = h_ref[...] * e
    contrib = jax.lax.dot_general(he, Pf, (((0,), (0,)), ((), ())),
                                  preferred_element_type=jnp.float32)
    acc[...] = acc[...] * scale + contrib

    s = sden[...]
    out_ref[...] = jnp.where(s > 0.0, acc[...] / jnp.where(s > 0.0, s, 1.0),
                             0.0)


def _tc_pool(h, gate8, batch8):
    return pl.pallas_call(
        _pool_kernel,
        grid=(NBLK,),
        in_specs=[pl.BlockSpec((256, D), lambda i: (i, 0)),
                  pl.BlockSpec((256, T), lambda i: (i, 0)),
                  pl.BlockSpec((256, T), lambda i: (i, 0))],
        out_specs=pl.BlockSpec((D, G), lambda i: (0, 0)),
        out_shape=jax.ShapeDtypeStruct((D, G), jnp.float32),
        scratch_shapes=[pltpu.VMEM((D, G), jnp.float32),
                        pltpu.VMEM((1, G), jnp.float32),
                        pltpu.VMEM((1, G), jnp.float32)],
    )(h, gate8, batch8)


def kernel(params, x, edge_index, batch):
    n, t = x.shape
    e = edge_index.shape[1]

    xp = jnp.zeros((N_PAD, T), jnp.int32).at[:n].set(x.astype(jnp.int32))
    xflat = xp.reshape(-1)
    src = jnp.zeros((E_PAD,), jnp.int32).at[:e].set(
        edge_index[0].astype(jnp.int32))
    dst = jnp.full((E_PAD,), N_PAD, jnp.int32).at[:e].set(
        edge_index[1].astype(jnp.int32))
    batch8 = jnp.broadcast_to(
        jnp.full((N_PAD,), G, jnp.int32).at[:n].set(
            batch.astype(jnp.int32))[:, None], (N_PAD, T))

    loc2, src2 = _tc_prep(dst.reshape(E_PAD // 128, 128),
                          src.reshape(E_PAD // 128, 128))
    loc2 = loc2.reshape(NC, E_PAD)
    src2 = src2.reshape(2, E_PAD)
    tokrows, degp = _sc_pre(params["emb"], xflat, loc2)
    tok3 = tokrows.reshape(N_PAD, T, D)

    DH = D // 2

    def msgpass(hN2):
        flat = _sc_msgpass(hN2.reshape(2 * N_PAD, DH), src2, loc2)
        return flat.reshape(2, N_PAD, DH)

    hNin, res0, norm8 = _tc_encoder(
        tok3, xp, degp, params["st_W"], params["st_b"], params["in_W"],
        params["res_W"], params["res_b"])

    sums = msgpass(hNin)
    h, hN = _tc_post0(sums, norm8, res0, params["in_b"], params["gcn_W"][0])

    nl = len(params["gcn_W"])
    for i in range(nl):
        sums = msgpass(hN)
        if i + 1 < nl:
            h, hN = _tc_layer(sums, norm8, h, params["gcn_b"][i],
                              params["ln_g"][i], params["ln_b"][i],
                              params["gcn_W"][i + 1])
        else:
            h, gate8 = _tc_last(sums, norm8, h, params["gcn_b"][i],
                                params["ln_g"][i], params["ln_b"][i],
                                params["gate_W1"], params["gate_b1"],
                                params["gate_W2"], params["gate_b2"])

    outT = _tc_pool(h, gate8, batch8)
    return outT.T


# speedup vs baseline: 2.4669x; 1.0921x over previous
import functools

import jax
import jax.numpy as jnp
from jax import lax
from jax.experimental import pallas as pl
from jax.experimental.pallas import tpu as pltpu
from jax.experimental.pallas import tpu_sc as plsc

NC, NS = 2, 16
NW = NC * NS
L = 16

N_PAD = 10240
E_PAD = 163840
HALF = N_PAD // 2
ACC_ROWS = HALF + 8
DEG_ROWS = 10496
T = 8
D = 256
G = 128
NBLK = N_PAD // 256
K = 128

_mesh = functools.partial(
    plsc.VectorSubcoreMesh, core_axis_name="c", subcore_axis_name="s")


def _sc_pre(emb, xflat, loc2):
    TOK = xflat.shape[0]
    tpw = TOK // NW
    ept = E_PAD // NS
    DACC = HALF + 128
    zrows = DACC // NS
    orows = HALF // NS
    DW = 128

    @functools.partial(
        pl.kernel,
        mesh=_mesh(),
        out_type=(jax.ShapeDtypeStruct((TOK, D), jnp.float32),
                  jax.ShapeDtypeStruct((NC * HALF, DW), jnp.float32)),
        scratch_types=[pltpu.VMEM((K,), jnp.int32),
                       pltpu.VMEM((K, D), jnp.float32),
                       pltpu.VMEM((K,), jnp.int32),
                       pltpu.VMEM((K, DW), jnp.float32),
                       pltpu.VMEM((41, DW), jnp.float32),
                       pltpu.VMEM_SHARED((DACC, DW), jnp.float32),
                       pltpu.SemaphoreType.DMA],
    )
    def k(emb_h, xf_h, loc_h, tok_h, degp_h, idxv, rows, locv, vbuf, zbuf,
          acc, sem):
        c = lax.axis_index("c")
        s = lax.axis_index("s")
        w = c * NS + s

        one16 = jnp.where(lax.iota(jnp.int32, L) == 0, 1.0, 0.0)
        zero16 = jnp.zeros((L,), jnp.float32)

        @pl.loop(0, K)
        def _fill(i):
            for l in range(DW // L):
                vbuf[i, pl.ds(l * L, L)] = one16 if l == 0 else zero16

        @pl.loop(0, 41)
        def _zfill(i):
            for l in range(DW // L):
                zbuf[i, pl.ds(l * L, L)] = zero16

        for z in range(zrows // 41):
            pltpu.sync_copy(zbuf, acc.at[pl.ds(s * zrows + z * 41, 41)])
        plsc.subcore_barrier()

        eb = s * ept

        @pl.loop(0, ept // K)
        def _deg(j):
            pltpu.sync_copy(loc_h.at[c, pl.ds(eb + j * K, K)], locv)
            pltpu.sync_copy(vbuf, acc.at[locv], add=True)

        tb = w * tpw

        @pl.loop(0, tpw // K)
        def _emb(j):
            base = tb + j * K
            pltpu.sync_copy(xf_h.at[pl.ds(base, K)], idxv)
            pltpu.async_copy(emb_h.at[idxv], rows, sem).wait()
            pltpu.sync_copy(rows, tok_h.at[pl.ds(base, K), :])

        plsc.subcore_barrier()
        pltpu.sync_copy(acc.at[pl.ds(s * orows, orows)],
                        degp_h.at[pl.ds(c * HALF + s * orows, orows), :])

    return k(emb, xflat, loc2)


def _sc_msgpass(hN2, src2, loc2):
    DH = D // 2
    ept = E_PAD // NS
    rpt = HALF // NS

    NB = 4
    GROUPS = (ept // K) // NB

    scratch = ([pltpu.VMEM((K,), jnp.int32) for _ in range(NB)] +
               [pltpu.VMEM((K,), jnp.int32) for _ in range(NB)] +
               [pltpu.VMEM((K, DH), jnp.float32) for _ in range(NB)] +
               [pltpu.VMEM_SHARED((ACC_ROWS, DH), jnp.float32)] +
               [pltpu.SemaphoreType.DMA for _ in range(3 * NB)])

    @functools.partial(
        pl.kernel,
        mesh=_mesh(),
        out_type=jax.ShapeDtypeStruct((2 * N_PAD, DH), jnp.float32),
        scratch_types=scratch,
    )
    def k(hN_h, src_h, loc_h, out_h, *rest):
        srcb = rest[0:NB]
        locb = rest[NB:2 * NB]
        rowsb = rest[2 * NB:3 * NB]
        acc = rest[3 * NB]
        semi = rest[3 * NB + 1:3 * NB + 1 + NB]
        semg = rest[3 * NB + 1 + NB:3 * NB + 1 + 2 * NB]
        sems = rest[3 * NB + 1 + 2 * NB:3 * NB + 1 + 3 * NB]

        c = lax.axis_index("c")
        s = lax.axis_index("s")
        gbase = c * HALF + s * rpt
        eb = s * ept

        for f in range(2):
            fb = f * N_PAD
            pltpu.sync_copy(hN_h.at[pl.ds(fb + gbase, rpt), :],
                            acc.at[pl.ds(s * rpt, rpt)])
            plsc.subcore_barrier()

            @pl.loop(0, GROUPS)
            def _edges(g):
                di = []
                for i in range(NB):
                    b = eb + (g * NB + i) * K
                    di.append(pltpu.async_copy(
                        src_h.at[f, pl.ds(b, K)], srcb[i], semi[i]))
                    di.append(pltpu.async_copy(
                        loc_h.at[c, pl.ds(b, K)], locb[i], semi[i]))
                for d in di:
                    d.wait()
                dg = [pltpu.async_copy(hN_h.at[srcb[i]], rowsb[i], semg[i])
                      for i in range(NB)]
                for d in dg:
                    d.wait()
                ds_ = [pltpu.async_copy(rowsb[i], acc.at[locb[i]], sems[i],
                                        add=True)
                       for i in range(NB)]
                for d in ds_:
                    d.wait()

            plsc.subcore_barrier()
            pltpu.sync_copy(acc.at[pl.ds(s * rpt, rpt)],
                            out_h.at[pl.ds(fb + gbase, rpt), :])

    return k(hN2, src2, loc2)


def _dot(a, b):
    return jax.lax.dot_general(a, b, (((1,), (0,)), ((), ())),
                               preferred_element_type=jnp.float32)


def _prep_kernel(dst_ref, src_ref, loc_ref, src2_ref):
    d = dst_ref[...]
    sv = src_ref[...]
    for c in range(NC):
        lo = c * HALF
        inr = jnp.logical_and(d >= lo, d < lo + HALF)
        loc_ref[c] = jnp.where(inr, d - lo, HALF)
    for f in range(2):
        src2_ref[f] = sv + f * N_PAD


def _tc_prep(dst2d, src2d):
    return pl.pallas_call(
        _prep_kernel,
        out_shape=[jax.ShapeDtypeStruct((NC,) + dst2d.shape, jnp.int32),
                   jax.ShapeDtypeStruct((2,) + dst2d.shape, jnp.int32)],
    )(dst2d, src2d)


def _enc_kernel(tok_ref, x_ref, degp_ref, stw_ref, stb_ref, inw_ref,
                resw_ref, resb_ref, hNin_ref, res0_ref, norm8_ref):
    xb = x_ref[...]
    mask = (xb != 0).astype(jnp.float32)
    tok = tok_ref[...]
    msum = jnp.sum(tok * mask[:, :, None], axis=1)
    cnt = jnp.sum(mask, axis=1, keepdims=True)
    node = msum / jnp.maximum(cnt, 1.0)
    node = _dot(node, stw_ref[...]) + stb_ref[...]

    deg = degp_ref[:, :1] + 1.0
    norm = jax.lax.rsqrt(jnp.maximum(deg, 1.0))
    norm8_ref[...] = jnp.broadcast_to(norm, norm8_ref.shape)

    hNin = _dot(node, inw_ref[...]) * norm
    hNin_ref[0] = hNin[:, :D // 2]
    hNin_ref[1] = hNin[:, D // 2:]
    res0_ref[...] = _dot(node, resw_ref[...]) + resb_ref[...]


def _tc_encoder(tok3, xp, degp, st_W, st_b, in_W, res_W, res_b):
    full = lambda shape: pl.BlockSpec(shape, lambda i: (0,) * len(shape))
    return pl.pallas_call(
        _enc_kernel,
        grid=(NBLK,),
        in_specs=[
            pl.BlockSpec((256, T, D), lambda i: (i, 0, 0)),
            pl.BlockSpec((256, T), lambda i: (i, 0)),
            pl.BlockSpec((256, 128), lambda i: (i, 0)),
            full((D, D)), full((1, D)), full((D, D)), full((D, D)),
            full((1, D)),
        ],
        out_specs=[
            pl.BlockSpec((2, 256, D // 2), lambda i: (0, i, 0)),
            pl.BlockSpec((256, D), lambda i: (i, 0)),
            pl.BlockSpec((256, T), lambda i: (i, 0)),
        ],
        out_shape=[
            jax.ShapeDtypeStruct((2, N_PAD, D // 2), jnp.float32),
            jax.ShapeDtypeStruct((N_PAD, D), jnp.float32),
            jax.ShapeDtypeStruct((N_PAD, T), jnp.float32),
        ],
    )(tok3, xp, degp, st_W, st_b.reshape(1, D), in_W, res_W,
      res_b.reshape(1, D))


def _cat(s2):
    return jnp.concatenate([s2[0], s2[1]], axis=1)


def _split_store(ref, v):
    ref[0] = v[:, :D // 2]
    ref[1] = v[:, D // 2:]


def _post0_kernel(sums_ref, norm8_ref, res0_ref, inb_ref, w_ref,
                  h_ref, hN_ref):
    nrm = norm8_ref[:, :1]
    h = jax.nn.relu(nrm * _cat(sums_ref[...]) + inb_ref[...]) + res0_ref[...]
    h_ref[...] = h
    _split_store(hN_ref, _dot(h, w_ref[...]) * nrm)


_sums_blk = lambda: pl.BlockSpec((2, 256, D // 2), lambda i: (0, i, 0))
_hN_shape = lambda: jax.ShapeDtypeStruct((2, N_PAD, D // 2), jnp.float32)


def _tc_post0(sums2, norm8, res0, in_b, W0):
    full = lambda shape: pl.BlockSpec(shape, lambda i: (0,) * len(shape))
    blk = pl.BlockSpec((256, D), lambda i: (i, 0))
    return pl.pallas_call(
        _post0_kernel,
        grid=(NBLK,),
        in_specs=[_sums_blk(), pl.BlockSpec((256, T), lambda i: (i, 0)), blk,
                  full((1, D)), full((D, D))],
        out_specs=[blk, _sums_blk()],
        out_shape=[jax.ShapeDtypeStruct((N_PAD, D), jnp.float32),
                   _hN_shape()],
    )(sums2, norm8, res0, in_b.reshape(1, D), W0)


def _layer_body(sums_ref, norm8_ref, hprev_ref, b_ref, g_ref, beta_ref):
    nrm = norm8_ref[:, :1]
    u = nrm * _cat(sums_ref[...]) + b_ref[...]
    mu = jnp.mean(u, axis=1, keepdims=True)
    var = jnp.mean(jnp.square(u - mu), axis=1, keepdims=True)
    u = (u - mu) * jax.lax.rsqrt(var + 1e-5) * g_ref[...] + beta_ref[...]
    return jax.nn.relu(u) + hprev_ref[...], nrm


def _layer_kernel(sums_ref, norm8_ref, hprev_ref, b_ref, g_ref, beta_ref,
                  w_ref, h_ref, hN_ref):
    h, nrm = _layer_body(sums_ref, norm8_ref, hprev_ref, b_ref, g_ref,
                         beta_ref)
    h_ref[...] = h
    _split_store(hN_ref, _dot(h, w_ref[...]) * nrm)


def _tc_layer(sums2, norm8, hprev, b, g, beta, Wnext):
    full = lambda shape: pl.BlockSpec(shape, lambda i: (0,) * len(shape))
    blk = pl.BlockSpec((256, D), lambda i: (i, 0))
    return pl.pallas_call(
        _layer_kernel,
        grid=(NBLK,),
        in_specs=[_sums_blk(), pl.BlockSpec((256, T), lambda i: (i, 0)), blk,
                  full((1, D)), full((1, D)), full((1, D)), full((D, D))],
        out_specs=[blk, _sums_blk()],
        out_shape=[jax.ShapeDtypeStruct((N_PAD, D), jnp.float32),
                   _hN_shape()],
    )(sums2, norm8, hprev, b.reshape(1, D), g.reshape(1, D),
      beta.reshape(1, D), Wnext)


def _last_kernel(sums_ref, norm8_ref, hprev_ref, b_ref, g_ref, beta_ref,
                 gw1_ref, gb1_ref, gw2_ref, gb2_ref, h_ref, gate8_ref):
    h, _ = _layer_body(sums_ref, norm8_ref, hprev_ref, b_ref, g_ref,
                       beta_ref)
    h_ref[...] = h
    g1 = jax.nn.relu(_dot(h, gw1_ref[...]) + gb1_ref[...])
    gate8_ref[...] = _dot(g1, gw2_ref[...]) + gb2_ref[...]


def _tc_last(sums2, norm8, hprev, b, g, beta, gw1, gb1, gw2, gb2):
    full = lambda shape: pl.BlockSpec(shape, lambda i: (0,) * len(shape))
    blk = pl.BlockSpec((256, D), lambda i: (i, 0))
    return pl.pallas_call(
        _last_kernel,
        grid=(NBLK,),
        in_specs=[_sums_blk(), pl.BlockSpec((256, T), lambda i: (i, 0)), blk,
                  full((1, D)), full((1, D)), full((1, D)),
                  full((D, D // 2)), full((1, D // 2)),
                  full((D // 2, T)), full((1, T))],
        out_specs=[blk, pl.BlockSpec((256, T), lambda i: (i, 0))],
        out_shape=[jax.ShapeDtypeStruct((N_PAD, D), jnp.float32),
                   jax.ShapeDtypeStruct((N_PAD, T), jnp.float32)],
    )(sums2, norm8, hprev, b.reshape(1, D), g.reshape(1, D),
      beta.reshape(1, D), gw1, gb1.reshape(1, D // 2),
      jnp.pad(gw2, ((0, 0), (0, T - 1))),
      jnp.broadcast_to(gb2.reshape(1, 1), (1, T)))


def _pool_kernel(h_ref, gate8_ref, batch8_ref, out_ref, acc, m, sden):
    i = pl.program_id(0)

    @pl.when(i == 0)
    def _init():
        acc[...] = jnp.zeros_like(acc)
        m[...] = jnp.full_like(m, -jnp.inf)
        sden[...] = jnp.zeros_like(sden)

    batch = batch8_ref[:, :1]
    gate = gate8_ref[:, :1]
    giota = lax.broadcasted_iota(jnp.int32, (256, G), 1)
    P = batch == giota
    Pf = P.astype(jnp.float32)

    m_old = m[...]
    cand = jnp.max(jnp.where(P, gate, -jnp.inf), axis=0, keepdims=True)
    m_new = jnp.maximum(m_old, cand)
    m[...] = m_new

    finite = m_new > -jnp.inf
    scale = jnp.exp(jnp.where(finite, m_old - m_new, 0.0))

    m_node = jnp.sum(jnp.where(P, jnp.where(finite, m_new, 0.0), 0.0),
                     axis=1, keepdims=True)
    valid = batch < G
    e = jnp.where(valid, jnp.exp(gate - m_node), 0.0)

    sden[...] = sden[...] * scale + jnp.sum(Pf * e, axis=0, keepdims=True)
    he = h_ref[...] * e
    contrib = jax.lax.dot_general(he, Pf, (((0,), (0,)), ((), ())),
                                  preferred_element_type=jnp.float32)
    acc[...] = acc[...] * scale + contrib

    s = sden[...]
    out_ref[...] = jnp.where(s > 0.0, acc[...] / jnp.where(s > 0.0, s, 1.0),
                             0.0)


def _tc_pool(h, gate8, batch8):
    return pl.pallas_call(
        _pool_kernel,
        grid=(NBLK,),
        in_specs=[pl.BlockSpec((256, D), lambda i: (i, 0)),
                  pl.BlockSpec((256, T), lambda i: (i, 0)),
                  pl.BlockSpec((256, T), lambda i: (i, 0))],
        out_specs=pl.BlockSpec((D, G), lambda i: (0, 0)),
        out_shape=jax.ShapeDtypeStruct((D, G), jnp.float32),
        scratch_shapes=[pltpu.VMEM((D, G), jnp.float32),
                        pltpu.VMEM((1, G), jnp.float32),
                        pltpu.VMEM((1, G), jnp.float32)],
    )(h, gate8, batch8)


def kernel(params, x, edge_index, batch):
    n, t = x.shape
    e = edge_index.shape[1]

    xp = jnp.zeros((N_PAD, T), jnp.int32).at[:n].set(x.astype(jnp.int32))
    xflat = xp.reshape(-1)
    src = jnp.zeros((E_PAD,), jnp.int32).at[:e].set(
        edge_index[0].astype(jnp.int32))
    dst = jnp.full((E_PAD,), N_PAD, jnp.int32).at[:e].set(
        edge_index[1].astype(jnp.int32))
    batch8 = jnp.broadcast_to(
        jnp.full((N_PAD,), G, jnp.int32).at[:n].set(
            batch.astype(jnp.int32))[:, None], (N_PAD, T))

    loc2, src2 = _tc_prep(dst.reshape(E_PAD // 128, 128),
                          src.reshape(E_PAD // 128, 128))
    loc2 = loc2.reshape(NC, E_PAD)
    src2 = src2.reshape(2, E_PAD)
    tokrows, degp = _sc_pre(params["emb"], xflat, loc2)
    tok3 = tokrows.reshape(N_PAD, T, D)

    DH = D // 2

    def msgpass(hN2):
        flat = _sc_msgpass(hN2.reshape(2 * N_PAD, DH), src2, loc2)
        return flat.reshape(2, N_PAD, DH)

    hNin, res0, norm8 = _tc_encoder(
        tok3, xp, degp, params["st_W"], params["st_b"], params["in_W"],
        params["res_W"], params["res_b"])

    sums = msgpass(hNin)
    h, hN = _tc_post0(sums, norm8, res0, params["in_b"], params["gcn_W"][0])

    nl = len(params["gcn_W"])
    for i in range(nl):
        sums = msgpass(hN)
        if i + 1 < nl:
            h, hN = _tc_layer(sums, norm8, h, params["gcn_b"][i],
                              params["ln_g"][i], params["ln_b"][i],
                              params["gcn_W"][i + 1])
        else:
            h, gate8 = _tc_last(sums, norm8, h, params["gcn_b"][i],
                                params["ln_g"][i], params["ln_b"][i],
                                params["gate_W1"], params["gate_b1"],
                                params["gate_W2"], params["gate_b2"])

    outT = _tc_pool(h, gate8, batch8)
    return outT.T


# speedup vs baseline: 2.4956x; 1.0117x over previous
import functools

import jax
import jax.numpy as jnp
from jax import lax
from jax.experimental import pallas as pl
from jax.experimental.pallas import tpu as pltpu
from jax.experimental.pallas import tpu_sc as plsc

NC, NS = 2, 16
NW = NC * NS
L = 16

N_PAD = 10240
E_PAD = 163840
HALF = N_PAD // 2
ACC_ROWS = HALF + 8
DEG_ROWS = 10496
T = 8
D = 256
G = 128
NBLK = N_PAD // 256
K = 128

_mesh = functools.partial(
    plsc.VectorSubcoreMesh, core_axis_name="c", subcore_axis_name="s")


def _sc_pre(emb, xflat, loc2):
    TOK = xflat.shape[0]
    tpw = TOK // NW
    ept = E_PAD // NS
    DACC = HALF + 128
    zrows = DACC // NS
    orows = HALF // NS
    DW = 128

    @functools.partial(
        pl.kernel,
        mesh=_mesh(),
        out_type=(jax.ShapeDtypeStruct((TOK, D), jnp.float32),
                  jax.ShapeDtypeStruct((NC * HALF, DW), jnp.float32)),
        scratch_types=[pltpu.VMEM((K,), jnp.int32),
                       pltpu.VMEM((K, D), jnp.float32),
                       pltpu.VMEM((K,), jnp.int32),
                       pltpu.VMEM((K, DW), jnp.float32),
                       pltpu.VMEM((41, DW), jnp.float32),
                       pltpu.VMEM_SHARED((DACC, DW), jnp.float32),
                       pltpu.SemaphoreType.DMA],
    )
    def k(emb_h, xf_h, loc_h, tok_h, degp_h, idxv, rows, locv, vbuf, zbuf,
          acc, sem):
        c = lax.axis_index("c")
        s = lax.axis_index("s")
        w = c * NS + s

        one16 = jnp.where(lax.iota(jnp.int32, L) == 0, 1.0, 0.0)
        zero16 = jnp.zeros((L,), jnp.float32)

        @pl.loop(0, K)
        def _fill(i):
            for l in range(DW // L):
                vbuf[i, pl.ds(l * L, L)] = one16 if l == 0 else zero16

        @pl.loop(0, 41)
        def _zfill(i):
            for l in range(DW // L):
                zbuf[i, pl.ds(l * L, L)] = zero16

        for z in range(zrows // 41):
            pltpu.sync_copy(zbuf, acc.at[pl.ds(s * zrows + z * 41, 41)])
        plsc.subcore_barrier()

        eb = s * ept

        @pl.loop(0, ept // K)
        def _deg(j):
            pltpu.sync_copy(loc_h.at[c, pl.ds(eb + j * K, K)], locv)
            pltpu.sync_copy(vbuf, acc.at[locv], add=True)

        tb = w * tpw

        @pl.loop(0, tpw // K)
        def _emb(j):
            base = tb + j * K
            pltpu.sync_copy(xf_h.at[pl.ds(base, K)], idxv)
            pltpu.async_copy(emb_h.at[idxv], rows, sem).wait()
            pltpu.sync_copy(rows, tok_h.at[pl.ds(base, K), :])

        plsc.subcore_barrier()
        pltpu.sync_copy(acc.at[pl.ds(s * orows, orows)],
                        degp_h.at[pl.ds(c * HALF + s * orows, orows), :])

    return k(emb, xflat, loc2)


def _sc_msgpass(hN2, src2, loc2):
    DH = D // 2
    ept = E_PAD // NS
    rpt = HALF // NS

    NB = 5
    GROUPS = (ept // K) // NB

    scratch = ([pltpu.VMEM((K,), jnp.int32) for _ in range(NB)] +
               [pltpu.VMEM((K,), jnp.int32) for _ in range(NB)] +
               [pltpu.VMEM((K, DH), jnp.float32) for _ in range(NB)] +
               [pltpu.VMEM_SHARED((ACC_ROWS, DH), jnp.float32)] +
               [pltpu.SemaphoreType.DMA for _ in range(3 * NB)])

    @functools.partial(
        pl.kernel,
        mesh=_mesh(),
        out_type=jax.ShapeDtypeStruct((2 * N_PAD, DH), jnp.float32),
        scratch_types=scratch,
    )
    def k(hN_h, src_h, loc_h, out_h, *rest):
        srcb = rest[0:NB]
        locb = rest[NB:2 * NB]
        rowsb = rest[2 * NB:3 * NB]
        acc = rest[3 * NB]
        semi = rest[3 * NB + 1:3 * NB + 1 + NB]
        semg = rest[3 * NB + 1 + NB:3 * NB + 1 + 2 * NB]
        sems = rest[3 * NB + 1 + 2 * NB:3 * NB + 1 + 3 * NB]

        c = lax.axis_index("c")
        s = lax.axis_index("s")
        gbase = c * HALF + s * rpt
        eb = s * ept

        for f in range(2):
            fb = f * N_PAD
            pltpu.sync_copy(hN_h.at[pl.ds(fb + gbase, rpt), :],
                            acc.at[pl.ds(s * rpt, rpt)])
            plsc.subcore_barrier()

            @pl.loop(0, GROUPS)
            def _edges(g):
                di = []
                for i in range(NB):
                    b = eb + (g * NB + i) * K
                    di.append(pltpu.async_copy(
                        src_h.at[f, pl.ds(b, K)], srcb[i], semi[i]))
                    di.append(pltpu.async_copy(
                        loc_h.at[c, pl.ds(b, K)], locb[i], semi[i]))
                dg = []
                for i in range(NB):
                    di[2 * i].wait()
                    di[2 * i + 1].wait()
                    dg.append(pltpu.async_copy(hN_h.at[srcb[i]], rowsb[i],
                                               semg[i]))
                ds_ = []
                for i in range(NB):
                    dg[i].wait()
                    ds_.append(pltpu.async_copy(rowsb[i], acc.at[locb[i]],
                                                sems[i], add=True))
                for d in ds_:
                    d.wait()

            plsc.subcore_barrier()
            pltpu.sync_copy(acc.at[pl.ds(s * rpt, rpt)],
                            out_h.at[pl.ds(fb + gbase, rpt), :])

    return k(hN2, src2, loc2)


def _dot(a, b):
    return jax.lax.dot_general(a, b, (((1,), (0,)), ((), ())),
                               preferred_element_type=jnp.float32)


def _prep_kernel(dst_ref, src_ref, loc_ref, src2_ref):
    d = dst_ref[...]
    sv = src_ref[...]
    for c in range(NC):
        lo = c * HALF
        inr = jnp.logical_and(d >= lo, d < lo + HALF)
        loc_ref[c] = jnp.where(inr, d - lo, HALF)
    for f in range(2):
        src2_ref[f] = sv + f * N_PAD


def _tc_prep(dst2d, src2d):
    return pl.pallas_call(
        _prep_kernel,
        out_shape=[jax.ShapeDtypeStruct((NC,) + dst2d.shape, jnp.int32),
                   jax.ShapeDtypeStruct((2,) + dst2d.shape, jnp.int32)],
    )(dst2d, src2d)


def _enc_kernel(tok_ref, x_ref, degp_ref, stw_ref, stb_ref, inw_ref,
                resw_ref, resb_ref, hNin_ref, res0_ref, norm8_ref):
    xb = x_ref[...]
    mask = (xb != 0).astype(jnp.float32)
    tok = tok_ref[...]
    msum = jnp.sum(tok * mask[:, :, None], axis=1)
    cnt = jnp.sum(mask, axis=1, keepdims=True)
    node = msum / jnp.maximum(cnt, 1.0)
    node = _dot(node, stw_ref[...]) + stb_ref[...]

    deg = degp_ref[:, :1] + 1.0
    norm = jax.lax.rsqrt(jnp.maximum(deg, 1.0))
    norm8_ref[...] = jnp.broadcast_to(norm, norm8_ref.shape)

    hNin = _dot(node, inw_ref[...]) * norm
    hNin_ref[0] = hNin[:, :D // 2]
    hNin_ref[1] = hNin[:, D // 2:]
    res0_ref[...] = _dot(node, resw_ref[...]) + resb_ref[...]


def _tc_encoder(tok3, xp, degp, st_W, st_b, in_W, res_W, res_b):
    full = lambda shape: pl.BlockSpec(shape, lambda i: (0,) * len(shape))
    return pl.pallas_call(
        _enc_kernel,
        grid=(NBLK,),
        in_specs=[
            pl.BlockSpec((256, T, D), lambda i: (i, 0, 0)),
            pl.BlockSpec((256, T), lambda i: (i, 0)),
            pl.BlockSpec((256, 128), lambda i: (i, 0)),
            full((D, D)), full((1, D)), full((D, D)), full((D, D)),
            full((1, D)),
        ],
        out_specs=[
            pl.BlockSpec((2, 256, D // 2), lambda i: (0, i, 0)),
            pl.BlockSpec((256, D), lambda i: (i, 0)),
            pl.BlockSpec((256, T), lambda i: (i, 0)),
        ],
        out_shape=[
            jax.ShapeDtypeStruct((2, N_PAD, D // 2), jnp.float32),
            jax.ShapeDtypeStruct((N_PAD, D), jnp.float32),
            jax.ShapeDtypeStruct((N_PAD, T), jnp.float32),
        ],
    )(tok3, xp, degp, st_W, st_b.reshape(1, D), in_W, res_W,
      res_b.reshape(1, D))


def _cat(s2):
    return jnp.concatenate([s2[0], s2[1]], axis=1)


def _split_store(ref, v):
    ref[0] = v[:, :D // 2]
    ref[1] = v[:, D // 2:]


def _post0_kernel(sums_ref, norm8_ref, res0_ref, inb_ref, w_ref,
                  h_ref, hN_ref):
    nrm = norm8_ref[:, :1]
    h = jax.nn.relu(nrm * _cat(sums_ref[...]) + inb_ref[...]) + res0_ref[...]
    h_ref[...] = h
    _split_store(hN_ref, _dot(h, w_ref[...]) * nrm)


_sums_blk = lambda: pl.BlockSpec((2, 256, D // 2), lambda i: (0, i, 0))
_hN_shape = lambda: jax.ShapeDtypeStruct((2, N_PAD, D // 2), jnp.float32)


def _tc_post0(sums2, norm8, res0, in_b, W0):
    full = lambda shape: pl.BlockSpec(shape, lambda i: (0,) * len(shape))
    blk = pl.BlockSpec((256, D), lambda i: (i, 0))
    return pl.pallas_call(
        _post0_kernel,
        grid=(NBLK,),
        in_specs=[_sums_blk(), pl.BlockSpec((256, T), lambda i: (i, 0)), blk,
                  full((1, D)), full((D, D))],
        out_specs=[blk, _sums_blk()],
        out_shape=[jax.ShapeDtypeStruct((N_PAD, D), jnp.float32),
                   _hN_shape()],
    )(sums2, norm8, res0, in_b.reshape(1, D), W0)


def _layer_body(sums_ref, norm8_ref, hprev_ref, b_ref, g_ref, beta_ref):
    nrm = norm8_ref[:, :1]
    u = nrm * _cat(sums_ref[...]) + b_ref[...]
    mu = jnp.mean(u, axis=1, keepdims=True)
    var = jnp.mean(jnp.square(u - mu), axis=1, keepdims=True)
    u = (u - mu) * jax.lax.rsqrt(var + 1e-5) * g_ref[...] + beta_ref[...]
    return jax.nn.relu(u) + hprev_ref[...], nrm


def _layer_kernel(sums_ref, norm8_ref, hprev_ref, b_ref, g_ref, beta_ref,
                  w_ref, h_ref, hN_ref):
    h, nrm = _layer_body(sums_ref, norm8_ref, hprev_ref, b_ref, g_ref,
                         beta_ref)
    h_ref[...] = h
    _split_store(hN_ref, _dot(h, w_ref[...]) * nrm)


def _tc_layer(sums2, norm8, hprev, b, g, beta, Wnext):
    full = lambda shape: pl.BlockSpec(shape, lambda i: (0,) * len(shape))
    blk = pl.BlockSpec((256, D), lambda i: (i, 0))
    return pl.pallas_call(
        _layer_kernel,
        grid=(NBLK,),
        in_specs=[_sums_blk(), pl.BlockSpec((256, T), lambda i: (i, 0)), blk,
                  full((1, D)), full((1, D)), full((1, D)), full((D, D))],
        out_specs=[blk, _sums_blk()],
        out_shape=[jax.ShapeDtypeStruct((N_PAD, D), jnp.float32),
                   _hN_shape()],
    )(sums2, norm8, hprev, b.reshape(1, D), g.reshape(1, D),
      beta.reshape(1, D), Wnext)


def _last_kernel(sums_ref, norm8_ref, hprev_ref, b_ref, g_ref, beta_ref,
                 gw1_ref, gb1_ref, gw2_ref, gb2_ref, h_ref, gate8_ref):
    h, _ = _layer_body(sums_ref, norm8_ref, hprev_ref, b_ref, g_ref,
                       beta_ref)
    h_ref[...] = h
    g1 = jax.nn.relu(_dot(h, gw1_ref[...]) + gb1_ref[...])
    gate8_ref[...] = _dot(g1, gw2_ref[...]) + gb2_ref[...]


def _tc_last(sums2, norm8, hprev, b, g, beta, gw1, gb1, gw2, gb2):
    full = lambda shape: pl.BlockSpec(shape, lambda i: (0,) * len(shape))
    blk = pl.BlockSpec((256, D), lambda i: (i, 0))
    return pl.pallas_call(
        _last_kernel,
        grid=(NBLK,),
        in_specs=[_sums_blk(), pl.BlockSpec((256, T), lambda i: (i, 0)), blk,
                  full((1, D)), full((1, D)), full((1, D)),
                  full((D, D // 2)), full((1, D // 2)),
                  full((D // 2, T)), full((1, T))],
        out_specs=[blk, pl.BlockSpec((256, T), lambda i: (i, 0))],
        out_shape=[jax.ShapeDtypeStruct((N_PAD, D), jnp.float32),
                   jax.ShapeDtypeStruct((N_PAD, T), jnp.float32)],
    )(sums2, norm8, hprev, b.reshape(1, D), g.reshape(1, D),
      beta.reshape(1, D), gw1, gb1.reshape(1, D // 2),
      jnp.pad(gw2, ((0, 0), (0, T - 1))),
      jnp.broadcast_to(gb2.reshape(1, 1), (1, T)))


def _pool_kernel(h_ref, gate8_ref, batch8_ref, out_ref, acc, m, sden):
    i = pl.program_id(0)

    @pl.when(i == 0)
    def _init():
        acc[...] = jnp.zeros_like(acc)
        m[...] = jnp.full_like(m, -jnp.inf)
        sden[...] = jnp.zeros_like(sden)

    batch = batch8_ref[:, :1]
    gate = gate8_ref[:, :1]
    giota = lax.broadcasted_iota(jnp.int32, (256, G), 1)
    P = batch == giota
    Pf = P.astype(jnp.float32)

    m_old = m[...]
    cand = jnp.max(jnp.where(P, gate, -jnp.inf), axis=0, keepdims=True)
    m_new = jnp.maximum(m_old, cand)
    m[...] = m_new

    finite = m_new > -jnp.inf
    scale = jnp.exp(jnp.where(finite, m_old - m_new, 0.0))

    m_node = jnp.sum(jnp.where(P, jnp.where(finite, m_new, 0.0), 0.0),
                     axis=1, keepdims=True)
    valid = batch < G
    e = jnp.where(valid, jnp.exp(gate - m_node), 0.0)

    sden[...] = sden[...] * scale + jnp.sum(Pf * e, axis=0, keepdims=True)
    he = h_ref[...] * e
    contrib = jax.lax.dot_general(he, Pf, (((0,), (0,)), ((), ())),
                                  preferred_element_type=jnp.float32)
    acc[...] = acc[...] * scale + contrib

    s = sden[...]
    out_ref[...] = jnp.where(s > 0.0, acc[...] / jnp.where(s > 0.0, s, 1.0),
                             0.0)


def _tc_pool(h, gate8, batch8):
    return pl.pallas_call(
        _pool_kernel,
        grid=(NBLK,),
        in_specs=[pl.BlockSpec((256, D), lambda i: (i, 0)),
                  pl.BlockSpec((256, T), lambda i: (i, 0)),
                  pl.BlockSpec((256, T), lambda i: (i, 0))],
        out_specs=pl.BlockSpec((D, G), lambda i: (0, 0)),
        out_shape=jax.ShapeDtypeStruct((D, G), jnp.float32),
        scratch_shapes=[pltpu.VMEM((D, G), jnp.float32),
                        pltpu.VMEM((1, G), jnp.float32),
                        pltpu.VMEM((1, G), jnp.float32)],
    )(h, gate8, batch8)


def kernel(params, x, edge_index, batch):
    n, t = x.shape
    e = edge_index.shape[1]

    xp = jnp.zeros((N_PAD, T), jnp.int32).at[:n].set(x.astype(jnp.int32))
    xflat = xp.reshape(-1)
    src = jnp.zeros((E_PAD,), jnp.int32).at[:e].set(
        edge_index[0].astype(jnp.int32))
    dst = jnp.full((E_PAD,), N_PAD, jnp.int32).at[:e].set(
        edge_index[1].astype(jnp.int32))
    batch8 = jnp.broadcast_to(
        jnp.full((N_PAD,), G, jnp.int32).at[:n].set(
            batch.astype(jnp.int32))[:, None], (N_PAD, T))

    loc2, src2 = _tc_prep(dst.reshape(E_PAD // 128, 128),
                          src.reshape(E_PAD // 128, 128))
    loc2 = loc2.reshape(NC, E_PAD)
    src2 = src2.reshape(2, E_PAD)
    tokrows, degp = _sc_pre(params["emb"], xflat, loc2)
    tok3 = tokrows.reshape(N_PAD, T, D)

    DH = D // 2

    def msgpass(hN2):
        flat = _sc_msgpass(hN2.reshape(2 * N_PAD, DH), src2, loc2)
        return flat.reshape(2, N_PAD, DH)

    hNin, res0, norm8 = _tc_encoder(
        tok3, xp, degp, params["st_W"], params["st_b"], params["in_W"],
        params["res_W"], params["res_b"])

    sums = msgpass(hNin)
    h, hN = _tc_post0(sums, norm8, res0, params["in_b"], params["gcn_W"][0])

    nl = len(params["gcn_W"])
    for i in range(nl):
        sums = msgpass(hN)
        if i + 1 < nl:
            h, hN = _tc_layer(sums, norm8, h, params["gcn_b"][i],
                              params["ln_g"][i], params["ln_b"][i],
                              params["gcn_W"][i + 1])
        else:
            h, gate8 = _tc_last(sums, norm8, h, params["gcn_b"][i],
                                params["ln_g"][i], params["ln_b"][i],
                                params["gate_W1"], params["gate_b1"],
                                params["gate_W2"], params["gate_b2"])

    outT = _tc_pool(h, gate8, batch8)
    return outT.T
